# Initial kernel scaffold; baseline (speedup 1.0000x reference)
import jax
import jax.numpy as jnp
from jax.experimental import pallas as pl


def kernel(x, pos_emb):
    raise NotImplementedError("write your pallas kernel here")


# speedup vs baseline: 13.7918x; 13.7918x over previous
import jax
import jax.numpy as jnp
from jax.experimental import pallas as pl

_BATCH_BLOCK = 256


def _broadcast_kernel(emb_ref, out_ref):
    out_ref[...] = jnp.broadcast_to(emb_ref[...], out_ref.shape)


def kernel(x, pos_emb):
    batch = x.shape[0]
    seq, emb = pos_emb.shape
    flat = seq * emb
    table = pos_emb.reshape(1, flat)
    blk = _BATCH_BLOCK
    while batch % blk:
        blk //= 2
    out = pl.pallas_call(
        _broadcast_kernel,
        grid=(batch // blk,),
        in_specs=[pl.BlockSpec((1, flat), lambda i: (0, 0))],
        out_specs=pl.BlockSpec((blk, flat), lambda i: (i, 0)),
        out_shape=jax.ShapeDtypeStruct((batch, flat), pos_emb.dtype),
    )(table)
    return out.reshape(batch, seq, emb)
